# baseline (speedup 1.0000x reference)
import functools

import jax
import jax.numpy as jnp
from jax import lax
from jax.experimental import pallas as pl
from jax.experimental.pallas import tpu as pltpu
from jax.experimental.pallas import tpu_sc as plsc

BATCH = 16384
D = 64
L = 16
NC = 2
NS = 16
NW = NC * NS
CHUNK = BATCH // NW
W = 128
NSUB = CHUNK // W


def _dme_body(s_hbm, r_hbm, o_hbm, edm_hbm, rdm_hbm, e_hbm, rh_hbm, rt_hbm,
              out_hbm,
              s_v, r_v, o_v, se_v, oe_v, sdm_v, odm_v, rh_v, rt_v, rdm_v,
              tmp_v, out_v, sem):
    wid = lax.axis_index("s") * NC + lax.axis_index("c")
    base0 = wid * CHUNK
    for sub in range(NSUB):
        base = base0 + sub * W
        pltpu.sync_copy(s_hbm.at[pl.ds(base, W)], s_v)
        pltpu.sync_copy(r_hbm.at[pl.ds(base, W)], r_v)
        pltpu.sync_copy(o_hbm.at[pl.ds(base, W)], o_v)
        copies = [
            pltpu.async_copy(e_hbm.at[s_v], se_v, sem),
            pltpu.async_copy(e_hbm.at[o_v], oe_v, sem),
            pltpu.async_copy(edm_hbm.at[s_v], sdm_v, sem),
            pltpu.async_copy(edm_hbm.at[o_v], odm_v, sem),
            pltpu.async_copy(rh_hbm.at[r_v], rh_v, sem),
            pltpu.async_copy(rt_hbm.at[r_v], rt_v, sem),
            pltpu.async_copy(rdm_hbm.at[r_v], rdm_v, sem),
        ]
        for c in copies:
            c.wait()

        iota = lax.iota(jnp.int32, L)

        def body(g, carry):
            for bl in range(L):
                b = g * L + bl
                acc = jnp.zeros((L,), jnp.float32)
                for k in range(D // L):
                    sl = pl.ds(k * L, L)
                    acc = (acc
                           + se_v[b, sl] * rh_v[b, sl]
                           + oe_v[b, sl] * rt_v[b, sl]
                           + sdm_v[b, sl] * rdm_v[b, sl] * odm_v[b, sl])
                tmp_v[pl.ds(bl * L, L)] = acc
            out16 = jnp.zeros((L,), jnp.float32)
            row_base = iota * L
            for j in range(L):
                col = plsc.load_gather(tmp_v, [row_base + j])
                out16 = out16 + col
            out_v[pl.ds(g * L, L)] = out16
            return carry

        lax.fori_loop(0, W // L, body, 0)
        pltpu.sync_copy(out_v, out_hbm.at[pl.ds(base, W)])


@jax.jit
def kernel(s, r, o, E_DM, R_DM, E, R_head, R_tail):
    si = s.astype(jnp.int32)
    ri = r.astype(jnp.int32)
    oi = o.astype(jnp.int32)
    run = pl.kernel(
        _dme_body,
        out_type=jax.ShapeDtypeStruct((BATCH,), jnp.float32),
        mesh=plsc.VectorSubcoreMesh(core_axis_name="c", subcore_axis_name="s"),
        compiler_params=pltpu.CompilerParams(
            needs_layout_passes=False, use_tc_tiling_on_sc=False),
        scratch_types=[
            pltpu.VMEM((W,), jnp.int32),
            pltpu.VMEM((W,), jnp.int32),
            pltpu.VMEM((W,), jnp.int32),
            pltpu.VMEM((W, D), jnp.float32),
            pltpu.VMEM((W, D), jnp.float32),
            pltpu.VMEM((W, D), jnp.float32),
            pltpu.VMEM((W, D), jnp.float32),
            pltpu.VMEM((W, D), jnp.float32),
            pltpu.VMEM((W, D), jnp.float32),
            pltpu.VMEM((W, D), jnp.float32),
            pltpu.VMEM((L * L,), jnp.float32),
            pltpu.VMEM((W,), jnp.float32),
            pltpu.SemaphoreType.DMA,
        ],
    )
    return run(si, ri, oi, E_DM, R_DM, E, R_head, R_tail)
